# baseline (speedup 1.0000x reference)
import functools

import jax
import jax.numpy as jnp
from jax import lax
from jax.experimental import pallas as pl
from jax.experimental.pallas import tpu as pltpu
from jax.experimental.pallas import tpu_sc as plsc

_ROWS = 64
_N = 8192
_K1 = 129
_L = 16
_NC = 2
_NS = 16
_NW = _NC * _NS
_RPW = _ROWS // _NW
_NB = 256
_CAP = 256

_mesh = plsc.VectorSubcoreMesh(core_axis_name="c", subcore_axis_name="s")


@functools.partial(
    pl.kernel,
    out_type=jax.ShapeDtypeStruct((_ROWS, _N), jnp.float32),
    mesh=_mesh,
    compiler_params=pltpu.CompilerParams(needs_layout_passes=False),
    scratch_types=[
        pltpu.VMEM((_RPW * _N,), jnp.float32),
        pltpu.VMEM((_RPW * _N,), jnp.float32),
        pltpu.VMEM((_L * _NB,), jnp.int32),
        pltpu.VMEM((_N + _L,), jnp.int32),
    ],
)
def _ksparse_kernel(x_hbm, out_hbm, rows_v, outs_v, hist_v, cand_v):
    wid = lax.axis_index("s") * _NC + lax.axis_index("c")
    base = wid * _RPW
    for r in range(_RPW):
        pltpu.sync_copy(
            x_hbm.at[base + r], rows_v.at[pl.ds(r * _N, _N)]
        )

    zero_i = jnp.zeros((_L,), jnp.int32)
    one_i = jnp.ones((_L,), jnp.int32)
    k1_v = jnp.full((_L,), _K1, jnp.int32)
    min_v = jnp.full((_L,), -(2 ** 31), jnp.int32)
    zero_f = jnp.zeros((_L,), jnp.float32)
    iota_i = lax.iota(jnp.int32, _L)
    lane_off = iota_i * jnp.int32(_NB)
    m31 = jnp.int32(0x7FFFFFFF)

    def to_key(x):
        b = lax.bitcast_convert_type(x, jnp.int32)
        return b ^ (lax.shift_right_arithmetic(b, 31) & m31)

    def row_body(r, _):
        rb = r * jnp.int32(_N)

        def zero_body(i):
            hist_v[pl.ds(i, _L)] = zero_i

        plsc.parallel_loop(0, _L * _NB, step=_L, unroll=8)(zero_body)

        def hist_body(i):
            k = to_key(rows_v[pl.ds(rb + i, _L)])
            ub = lax.shift_right_logical(k, 24) ^ jnp.int32(128)
            plsc.addupdate_scatter(hist_v, [lane_off + ub], one_i)

        plsc.parallel_loop(0, _N, step=_L, unroll=4)(hist_body)

        def scan_body(j, carry):
            above, bstar, done = carry
            c = jnp.int32(_L - 1) - j
            cb = c * jnp.int32(_L)
            t_c = hist_v[pl.ds(cb, _L)]
            for l in range(1, _L):
                t_c = t_c + hist_v[pl.ds(cb + l * _NB, _L)]
            r_c = lax.rev(plsc.cumsum(lax.rev(t_c, (0,))), (0,))
            s_c = r_c + above
            above = above + lax.broadcast(r_c[0], (_L,))
            pc = plsc.all_reduce_population_count(s_c >= k1_v)
            in_c = lax.broadcast(cb, (_L,)) + pc - one_i
            fresh = jnp.where(pc > zero_i, one_i, zero_i) * (one_i - done)
            bstar = jnp.where(fresh > zero_i, in_c, bstar)
            done = done | fresh
            return above, bstar, done

        _, bstar, _ = lax.fori_loop(
            0, _L, scan_body, (zero_i, zero_i, zero_i)
        )

        tv = lax.shift_left(bstar ^ jnp.full((_L,), 128, jnp.int32), 24)

        def fill_body(i):
            cand_v[pl.ds(i, _L)] = min_v

        plsc.parallel_loop(0, _CAP + _L, step=_L, unroll=8)(fill_body)

        def comp_body(i, off):
            k = to_key(rows_v[pl.ds(rb + i, _L)])
            m = k >= tv
            ones_m = jnp.where(m, one_i, zero_i)
            pos = off + plsc.cumsum(ones_m) - one_i
            plsc.store_scatter(cand_v, [pos], k, mask=m)
            return off + plsc.all_reduce_population_count(m)

        ncv = plsc.parallel_loop(0, _N, step=_L, unroll=8, carry=zero_i)(
            comp_body
        )
        plsc.store_scatter(cand_v, [ncv + iota_i], min_v)
        nc = ncv[0]
        nvec = lax.div(nc + jnp.int32(_L - 1), jnp.int32(_L))

        def fast_search(tv):
            wins = [cand_v[pl.ds(j * _L, _L)] for j in range(_CAP // _L)]
            for bit in reversed(range(24)):
                tent = tv + jnp.full((_L,), 1 << bit, jnp.int32)
                acc = zero_i
                for w in wins:
                    acc = acc + plsc.all_reduce_population_count(w >= tent)
                tv = jnp.where(acc >= k1_v, tent, tv)
            return tv

        def slow_search(tv):
            def count_c(tvec):
                def tbody(j, a):
                    k = cand_v[pl.ds(j * _L, _L)]
                    return a + plsc.all_reduce_population_count(k >= tvec)

                return lax.fori_loop(0, nvec, tbody, zero_i)

            def bit_lo(i, tv):
                bit_v = lax.broadcast(jnp.int32(23) - i, (_L,))
                tent = tv + lax.shift_left(one_i, bit_v)
                return jnp.where(count_c(tent) >= k1_v, tent, tv)

            return lax.fori_loop(0, 24, bit_lo, tv)

        tv = lax.cond(nc <= jnp.int32(_CAP), fast_search, slow_search, tv)

        thr_f = lax.bitcast_convert_type(
            tv ^ (lax.shift_right_arithmetic(tv, 31) & m31), jnp.float32
        )

        def mask_body(i):
            x = rows_v[pl.ds(rb + i, _L)]
            outs_v[pl.ds(rb + i, _L)] = jnp.where(x > thr_f, x, zero_f)

        plsc.parallel_loop(0, _N, step=_L, unroll=8)(mask_body)
        return 0

    lax.fori_loop(0, _RPW, row_body, 0)

    for r in range(_RPW):
        pltpu.sync_copy(
            outs_v.at[pl.ds(r * _N, _N)], out_hbm.at[base + r]
        )


def kernel(inputs):
    return _ksparse_kernel(inputs)


# speedup vs baseline: 1.0483x; 1.0483x over previous
import functools

import jax
import jax.numpy as jnp
from jax import lax
from jax.experimental import pallas as pl
from jax.experimental.pallas import tpu as pltpu
from jax.experimental.pallas import tpu_sc as plsc

_ROWS = 64
_N = 8192
_K1 = 129
_L = 16
_NC = 2
_NS = 16
_NW = _NC * _NS
_RPW = _ROWS // _NW
_NB = 256
_CAP = 256

_mesh = plsc.VectorSubcoreMesh(core_axis_name="c", subcore_axis_name="s")


@functools.partial(
    pl.kernel,
    out_type=jax.ShapeDtypeStruct((_ROWS, _N), jnp.float32),
    mesh=_mesh,
    compiler_params=pltpu.CompilerParams(needs_layout_passes=False),
    scratch_types=[
        pltpu.VMEM((_RPW * _N,), jnp.float32),
        pltpu.VMEM((_RPW * _N,), jnp.float32),
        pltpu.VMEM((_L * _NB,), jnp.int32),
        pltpu.VMEM((_N + _L,), jnp.int32),
        pltpu.SemaphoreType.DMA,
        pltpu.SemaphoreType.DMA,
    ],
)
def _ksparse_kernel(
    x_hbm, out_hbm, rows_v, outs_v, hist_v, cand_v, in_sem, out_sem
):
    wid = lax.axis_index("s") * _NC + lax.axis_index("c")
    base = wid * _RPW
    for r in range(_RPW):
        pltpu.async_copy(
            x_hbm.at[base + r], rows_v.at[pl.ds(r * _N, _N)], in_sem
        )
    for r in range(_RPW):
        pltpu.make_async_copy(
            x_hbm.at[base + r], rows_v.at[pl.ds(r * _N, _N)], in_sem
        ).wait()

    zero_i = jnp.zeros((_L,), jnp.int32)
    one_i = jnp.ones((_L,), jnp.int32)
    k1_v = jnp.full((_L,), _K1, jnp.int32)
    min_v = jnp.full((_L,), -(2 ** 31), jnp.int32)
    zero_f = jnp.zeros((_L,), jnp.float32)
    iota_i = lax.iota(jnp.int32, _L)
    lane_off = iota_i * jnp.int32(_NB)
    m31 = jnp.int32(0x7FFFFFFF)

    def to_key(x):
        b = lax.bitcast_convert_type(x, jnp.int32)
        return b ^ (lax.shift_right_arithmetic(b, 31) & m31)

    def row_body(r, _):
        rb = r * jnp.int32(_N)

        def zero_body(i):
            hist_v[pl.ds(i, _L)] = zero_i

        plsc.parallel_loop(0, _L * _NB, step=_L, unroll=8)(zero_body)

        def hist_body(i):
            k = to_key(rows_v[pl.ds(rb + i, _L)])
            ub = lax.shift_right_logical(k, 24) ^ jnp.int32(128)
            plsc.addupdate_scatter(hist_v, [lane_off + ub], one_i)

        plsc.parallel_loop(0, _N, step=_L, unroll=8)(hist_body)

        def scan_body(j, carry):
            above, bstar, done = carry
            c = jnp.int32(_L - 1) - j
            cb = c * jnp.int32(_L)
            t_c = hist_v[pl.ds(cb, _L)]
            for l in range(1, _L):
                t_c = t_c + hist_v[pl.ds(cb + l * _NB, _L)]
            r_c = lax.rev(plsc.cumsum(lax.rev(t_c, (0,))), (0,))
            s_c = r_c + above
            above = above + lax.broadcast(r_c[0], (_L,))
            pc = plsc.all_reduce_population_count(s_c >= k1_v)
            in_c = lax.broadcast(cb, (_L,)) + pc - one_i
            fresh = jnp.where(pc > zero_i, one_i, zero_i) * (one_i - done)
            bstar = jnp.where(fresh > zero_i, in_c, bstar)
            done = done | fresh
            return above, bstar, done

        _, bstar, _ = lax.fori_loop(
            0, _L, scan_body, (zero_i, zero_i, zero_i)
        )

        tv = lax.shift_left(bstar ^ jnp.full((_L,), 128, jnp.int32), 24)

        def fill_body(i):
            cand_v[pl.ds(i, _L)] = min_v

        plsc.parallel_loop(0, _CAP + _L, step=_L, unroll=8)(fill_body)

        def comp_body(i, off):
            k = to_key(rows_v[pl.ds(rb + i, _L)])
            m = k >= tv
            ones_m = jnp.where(m, one_i, zero_i)
            pos = off + plsc.cumsum(ones_m) - one_i
            plsc.store_scatter(cand_v, [pos], k, mask=m)
            return off + plsc.all_reduce_population_count(m)

        ncv = plsc.parallel_loop(0, _N, step=_L, unroll=8, carry=zero_i)(
            comp_body
        )
        plsc.store_scatter(cand_v, [ncv + iota_i], min_v)
        nvec = lax.div(ncv[0] + jnp.int32(_L - 1), jnp.int32(_L))

        def count_c(tvec):
            def sbody(i, acc):
                k = cand_v[pl.ds(i, _L)]
                return acc + plsc.all_reduce_population_count(k >= tvec)

            acc = plsc.parallel_loop(
                0, _CAP, step=_L, unroll=4, carry=zero_i
            )(sbody)

            def tbody(j, a):
                k = cand_v[pl.ds(j * _L, _L)]
                return a + plsc.all_reduce_population_count(k >= tvec)

            return lax.fori_loop(_CAP // _L, nvec, tbody, acc)

        def bit_lo(i, tv):
            bit_v = lax.broadcast(jnp.int32(23) - i, (_L,))
            tent = tv + lax.shift_left(one_i, bit_v)
            return jnp.where(count_c(tent) >= k1_v, tent, tv)

        tv = lax.fori_loop(0, 24, bit_lo, tv)

        thr_f = lax.bitcast_convert_type(
            tv ^ (lax.shift_right_arithmetic(tv, 31) & m31), jnp.float32
        )

        def mask_body(i):
            x = rows_v[pl.ds(rb + i, _L)]
            outs_v[pl.ds(rb + i, _L)] = jnp.where(x > thr_f, x, zero_f)

        plsc.parallel_loop(0, _N, step=_L, unroll=8)(mask_body)
        pltpu.async_copy(
            outs_v.at[pl.ds(rb, _N)], out_hbm.at[base + r], out_sem
        )
        return 0

    lax.fori_loop(0, _RPW, row_body, 0)

    for r in range(_RPW):
        pltpu.make_async_copy(
            outs_v.at[pl.ds(r * _N, _N)], out_hbm.at[base + r], out_sem
        ).wait()


def kernel(inputs):
    return _ksparse_kernel(inputs)


# speedup vs baseline: 1.0535x; 1.0050x over previous
import functools

import jax
import jax.numpy as jnp
from jax import lax
from jax.experimental import pallas as pl
from jax.experimental.pallas import tpu as pltpu
from jax.experimental.pallas import tpu_sc as plsc

_ROWS = 64
_N = 8192
_K1 = 129
_L = 16
_NC = 2
_NS = 16
_NW = _NC * _NS
_RPW = _ROWS // _NW
_NB = 256
_CAP = 256

_mesh = plsc.VectorSubcoreMesh(core_axis_name="c", subcore_axis_name="s")


@functools.partial(
    pl.kernel,
    out_type=jax.ShapeDtypeStruct((_ROWS, _N), jnp.float32),
    mesh=_mesh,
    compiler_params=pltpu.CompilerParams(needs_layout_passes=False),
    scratch_types=[
        pltpu.VMEM((_RPW * _N,), jnp.float32),
        pltpu.VMEM((_RPW * _N,), jnp.float32),
        pltpu.VMEM((_L * _NB,), jnp.int32),
        pltpu.VMEM((_N + _L,), jnp.int32),
        pltpu.SemaphoreType.DMA,
        pltpu.SemaphoreType.DMA,
    ],
)
def _ksparse_kernel(
    x_hbm, out_hbm, rows_v, outs_v, hist_v, cand_v, in_sem, out_sem
):
    wid = lax.axis_index("s") * _NC + lax.axis_index("c")
    base = wid * _RPW
    for r in range(_RPW):
        pltpu.async_copy(
            x_hbm.at[base + r], rows_v.at[pl.ds(r * _N, _N)], in_sem
        )
    for r in range(_RPW):
        pltpu.make_async_copy(
            x_hbm.at[base + r], rows_v.at[pl.ds(r * _N, _N)], in_sem
        ).wait()

    zero_i = jnp.zeros((_L,), jnp.int32)
    one_i = jnp.ones((_L,), jnp.int32)
    k1_v = jnp.full((_L,), _K1, jnp.int32)
    min_v = jnp.full((_L,), -(2 ** 31), jnp.int32)
    zero_f = jnp.zeros((_L,), jnp.float32)
    iota_i = lax.iota(jnp.int32, _L)
    lane_off = iota_i * jnp.int32(_NB)
    m31 = jnp.int32(0x7FFFFFFF)

    def to_key(x):
        b = lax.bitcast_convert_type(x, jnp.int32)
        return b ^ (lax.shift_right_arithmetic(b, 31) & m31)

    def row_body(r, _):
        rb = r * jnp.int32(_N)

        def zero_body(i):
            hist_v[pl.ds(i, _L)] = zero_i

        plsc.parallel_loop(0, _L * _NB, step=_L, unroll=8)(zero_body)

        def hist_body(i):
            k = to_key(rows_v[pl.ds(rb + i, _L)])
            ub = lax.shift_right_logical(k, 24) ^ jnp.int32(128)
            plsc.addupdate_scatter(hist_v, [lane_off + ub], one_i)

        plsc.parallel_loop(0, _N, step=_L, unroll=8)(hist_body)

        def scan_body(j, carry):
            above, bstar, done = carry
            c = jnp.int32(_L - 1) - j
            cb = c * jnp.int32(_L)
            t_c = hist_v[pl.ds(cb, _L)]
            for l in range(1, _L):
                t_c = t_c + hist_v[pl.ds(cb + l * _NB, _L)]
            r_c = lax.rev(plsc.cumsum(lax.rev(t_c, (0,))), (0,))
            s_c = r_c + above
            above = above + lax.broadcast(r_c[0], (_L,))
            pc = plsc.all_reduce_population_count(s_c >= k1_v)
            in_c = lax.broadcast(cb, (_L,)) + pc - one_i
            fresh = jnp.where(pc > zero_i, one_i, zero_i) * (one_i - done)
            bstar = jnp.where(fresh > zero_i, in_c, bstar)
            done = done | fresh
            return above, bstar, done

        _, bstar, _ = lax.fori_loop(
            0, _L, scan_body, (zero_i, zero_i, zero_i)
        )

        tv = lax.shift_left(bstar ^ jnp.full((_L,), 128, jnp.int32), 24)

        def fill_body(i):
            cand_v[pl.ds(i, _L)] = min_v

        plsc.parallel_loop(0, _CAP + _L, step=_L, unroll=8)(fill_body)

        def comp_body(i, off):
            k = to_key(rows_v[pl.ds(rb + i, _L)])
            m = k >= tv
            ones_m = jnp.where(m, one_i, zero_i)
            pos = off + plsc.cumsum(ones_m) - one_i
            plsc.store_scatter(cand_v, [pos], k, mask=m)
            return off + plsc.all_reduce_population_count(m)

        ncv = plsc.parallel_loop(0, _N, step=_L, unroll=8, carry=zero_i)(
            comp_body
        )
        plsc.store_scatter(cand_v, [ncv + iota_i], min_v)
        nvec = lax.div(ncv[0] + jnp.int32(_L - 1), jnp.int32(_L))

        two_i = jnp.full((_L,), 2, jnp.int32)
        three_i = jnp.full((_L,), 3, jnp.int32)

        def bit_lo(i, tv):
            pos_v = lax.broadcast(jnp.int32(22) - 2 * i, (_L,))
            t1 = tv + lax.shift_left(one_i, pos_v)
            t2 = tv + lax.shift_left(two_i, pos_v)
            t3 = tv + lax.shift_left(three_i, pos_v)

            def sbody(j, accs):
                a1, a2, a3 = accs
                k = cand_v[pl.ds(j, _L)]
                return (
                    a1 + plsc.all_reduce_population_count(k >= t1),
                    a2 + plsc.all_reduce_population_count(k >= t2),
                    a3 + plsc.all_reduce_population_count(k >= t3),
                )

            accs = plsc.parallel_loop(
                0, _CAP, step=_L, unroll=2, carry=(zero_i, zero_i, zero_i)
            )(sbody)

            def tbody(j, accs):
                a1, a2, a3 = accs
                k = cand_v[pl.ds(j * _L, _L)]
                return (
                    a1 + plsc.all_reduce_population_count(k >= t1),
                    a2 + plsc.all_reduce_population_count(k >= t2),
                    a3 + plsc.all_reduce_population_count(k >= t3),
                )

            c1, c2, c3 = lax.fori_loop(_CAP // _L, nvec, tbody, accs)
            tv = jnp.where(c1 >= k1_v, t1, tv)
            tv = jnp.where(c2 >= k1_v, t2, tv)
            tv = jnp.where(c3 >= k1_v, t3, tv)
            return tv

        tv = lax.fori_loop(0, 12, bit_lo, tv)

        thr_f = lax.bitcast_convert_type(
            tv ^ (lax.shift_right_arithmetic(tv, 31) & m31), jnp.float32
        )

        def mask_body(i):
            x = rows_v[pl.ds(rb + i, _L)]
            outs_v[pl.ds(rb + i, _L)] = jnp.where(x > thr_f, x, zero_f)

        plsc.parallel_loop(0, _N, step=_L, unroll=8)(mask_body)
        pltpu.async_copy(
            outs_v.at[pl.ds(rb, _N)], out_hbm.at[base + r], out_sem
        )
        return 0

    lax.fori_loop(0, _RPW, row_body, 0)

    for r in range(_RPW):
        pltpu.make_async_copy(
            outs_v.at[pl.ds(r * _N, _N)], out_hbm.at[base + r], out_sem
        ).wait()


def kernel(inputs):
    return _ksparse_kernel(inputs)
